# baseline (speedup 1.0000x reference)
import functools

import jax
import jax.numpy as jnp
from jax import lax
from jax.experimental import pallas as pl
from jax.experimental.pallas import tpu as pltpu
from jax.experimental.pallas import tpu_sc as plsc

_DIM = 32
_K = 8192
_N = 8192
_BT = 512
_NC = 2
_NS = 16
_NW = _NC * _NS
_BPW = _N // _NW


def _argmin_body(x_ref, cb_ref, idx_ref, minv_ref):
    xt = x_ref[...]
    cb = cb_ref[...]
    m2 = 2.0 * jnp.dot(xt, cb, preferred_element_type=jnp.float32)
    x2 = jnp.sum(xt * xt, axis=1, keepdims=True)
    c2 = jnp.sum(cb * cb, axis=0, keepdims=True)
    dist = (x2 - m2) + c2
    minv = jnp.min(dist, axis=1, keepdims=True)
    iota = lax.broadcasted_iota(jnp.int32, dist.shape, 1).astype(jnp.float32)
    idxf = jnp.min(jnp.where(dist == minv, iota, jnp.float32(_K)), axis=1)
    idx_ref[...] = idxf.astype(jnp.int32)
    minv_ref[...] = jnp.sum(minv).reshape(1, 1, 1)


@functools.cache
def _argmin_call():
    return pl.pallas_call(
        _argmin_body,
        grid=(_N // _BT,),
        in_specs=[
            pl.BlockSpec((_BT, _DIM), lambda i: (i, 0)),
            pl.BlockSpec((_DIM, _K), lambda i: (0, 0)),
        ],
        out_specs=[
            pl.BlockSpec((_BT,), lambda i: (i,)),
            pl.BlockSpec((1, 1, 1), lambda i: (i, 0, 0)),
        ],
        out_shape=[
            jax.ShapeDtypeStruct((_N,), jnp.int32),
            jax.ShapeDtypeStruct((_N // _BT, 1, 1), jnp.float32),
        ],
    )


def _gather_body(idx_hbm, tab_hbm, out_hbm, idx_v, rows_v, sem):
    wid = lax.axis_index("s") * _NC + lax.axis_index("c")
    base = wid * _BPW
    pltpu.sync_copy(idx_hbm.at[pl.ds(base, _BPW)], idx_v)
    pltpu.async_copy(tab_hbm.at[idx_v], rows_v, sem).wait()
    pltpu.sync_copy(rows_v, out_hbm.at[pl.ds(base, _BPW)])


@functools.cache
def _gather_call():
    return pl.kernel(
        _gather_body,
        out_type=jax.ShapeDtypeStruct((_N, _DIM), jnp.float32),
        mesh=plsc.VectorSubcoreMesh(core_axis_name="c", subcore_axis_name="s"),
        compiler_params=pltpu.CompilerParams(use_tc_tiling_on_sc=False),
        scratch_types=[
            pltpu.VMEM((_BPW,), jnp.int32),
            pltpu.VMEM((_BPW, _DIM), jnp.float32),
            pltpu.SemaphoreType.DMA,
        ],
    )


def kernel(x, codebook):
    b, l, d = x.shape
    xf = x.reshape(_N, _DIM)
    idx_flat, minv = _argmin_call()(xf, codebook)
    tab = codebook.T.reshape(_K, _DIM)
    out_flat = _gather_call()(idx_flat, tab)
    x_out = out_flat.reshape(b, l, d)
    inner_loss = jnp.sum(minv) * jnp.float32(1.0 / (_N * _DIM))
    return (x_out, idx_flat.reshape(b, l), inner_loss)


# speedup vs baseline: 1.0133x; 1.0133x over previous
import functools

import jax
import jax.numpy as jnp
from jax import lax
from jax.experimental import pallas as pl
from jax.experimental.pallas import tpu as pltpu
from jax.experimental.pallas import tpu_sc as plsc

_DIM = 32
_K = 8192
_N = 8192
_BT = 1024
_NC = 2
_NS = 16
_NW = _NC * _NS
_BPW = _N // _NW


def _argmin_body(x_ref, cb_ref, idx_ref, minv_ref):
    xt = x_ref[...]
    cb = cb_ref[...]
    m2 = 2.0 * jnp.dot(xt, cb, preferred_element_type=jnp.float32)
    x2 = jnp.sum(xt * xt, axis=1, keepdims=True)
    c2 = jnp.sum(cb * cb, axis=0, keepdims=True)
    dist = (x2 - m2) + c2
    minv = jnp.min(dist, axis=1, keepdims=True)
    iota = lax.broadcasted_iota(jnp.int32, dist.shape, 1).astype(jnp.float32)
    idxf = jnp.min(jnp.where(dist == minv, iota, jnp.float32(_K)), axis=1)
    idx_ref[...] = idxf.astype(jnp.int32)
    minv_ref[...] = jnp.sum(minv).reshape(1, 1, 1)


@functools.cache
def _argmin_call():
    return pl.pallas_call(
        _argmin_body,
        grid=(_N // _BT,),
        in_specs=[
            pl.BlockSpec((_BT, _DIM), lambda i: (i, 0)),
            pl.BlockSpec((_DIM, _K), lambda i: (0, 0)),
        ],
        out_specs=[
            pl.BlockSpec((_BT,), lambda i: (i,)),
            pl.BlockSpec((1, 1, 1), lambda i: (i, 0, 0)),
        ],
        out_shape=[
            jax.ShapeDtypeStruct((_N,), jnp.int32),
            jax.ShapeDtypeStruct((_N // _BT, 1, 1), jnp.float32),
        ],
    )


def _gather_body(idx_hbm, tab_hbm, out_hbm, idx_v, rows_v, sem):
    wid = lax.axis_index("s") * _NC + lax.axis_index("c")
    base = wid * _BPW
    pltpu.sync_copy(idx_hbm.at[pl.ds(base, _BPW)], idx_v)
    pltpu.async_copy(tab_hbm.at[idx_v], rows_v, sem).wait()
    pltpu.sync_copy(rows_v, out_hbm.at[pl.ds(base, _BPW)])


@functools.cache
def _gather_call():
    return pl.kernel(
        _gather_body,
        out_type=jax.ShapeDtypeStruct((_N, _DIM), jnp.float32),
        mesh=plsc.VectorSubcoreMesh(core_axis_name="c", subcore_axis_name="s"),
        compiler_params=pltpu.CompilerParams(use_tc_tiling_on_sc=False),
        scratch_types=[
            pltpu.VMEM((_BPW,), jnp.int32),
            pltpu.VMEM((_BPW, _DIM), jnp.float32),
            pltpu.SemaphoreType.DMA,
        ],
    )


def kernel(x, codebook):
    b, l, d = x.shape
    xf = x.reshape(_N, _DIM)
    idx_flat, minv = _argmin_call()(xf, codebook)
    tab = codebook.T.reshape(_K, _DIM)
    out_flat = _gather_call()(idx_flat, tab)
    x_out = out_flat.reshape(b, l, d)
    inner_loss = jnp.sum(minv) * jnp.float32(1.0 / (_N * _DIM))
    return (x_out, idx_flat.reshape(b, l), inner_loss)
